# baseline (speedup 1.0000x reference)
import functools

import jax
import jax.numpy as jnp
from jax import lax
from jax.experimental import pallas as pl
from jax.experimental.pallas import tpu as pltpu
from jax.experimental.pallas import tpu_sc as plsc

NC = 2
NS = 16
LANES = 16
NW = NC * NS


def _table_body(x_ref, w_ref, y_ref):
    y_ref[0] = jnp.dot(x_ref[...], w_ref[0], preferred_element_type=jnp.float32)


def _index_body(t_ref, s_ref, d_ref, g1_ref, g2_ref, *, n, r):
    t = t_ref[...]
    g1_ref[...] = t * n + s_ref[...]
    g2_ref[...] = (t + r) * n + d_ref[...]


def _combine_body(a_ref, b_ref, o_ref):
    o_ref[...] = a_ref[0] + b_ref[0]


def kernel(edge_index, edge_type, embeddings, weights, biases):
    N, H = embeddings.shape
    L = weights.shape[0]
    RD = weights.shape[1]
    R = RD // 2
    E = edge_type.shape[0]

    w_last = weights[L - 1]
    b_last = biases[L - 1]
    KA = H + 8
    x_aug = jnp.concatenate(
        [embeddings,
         jnp.ones((N, 1), jnp.float32),
         jnp.zeros((N, KA - H - 1), jnp.float32)], axis=1)
    w_aug = jnp.concatenate(
        [w_last,
         b_last[:, None, :],
         jnp.zeros((RD, KA - H - 1, H), jnp.float32)], axis=1)

    NB = 10
    BN = N // NB
    y = pl.pallas_call(
        _table_body,
        grid=(RD, NB),
        in_specs=[
            pl.BlockSpec((BN, KA), lambda rd, nb: (nb, 0)),
            pl.BlockSpec((1, KA, H), lambda rd, nb: (rd, 0, 0)),
        ],
        out_specs=pl.BlockSpec((1, BN, H), lambda rd, nb: (rd, nb, 0)),
        out_shape=jax.ShapeDtypeStruct((RD, N, H), jnp.float32),
    )(x_aug, w_aug)
    y_flat = y.reshape(RD * N, H)

    EB = E // 128
    t2 = edge_type.reshape(EB, 128)
    s2 = edge_index[0].reshape(EB, 128)
    d2 = edge_index[1].reshape(EB, 128)
    g1, g2 = pl.pallas_call(
        functools.partial(_index_body, n=N, r=R),
        out_shape=(jax.ShapeDtypeStruct((EB, 128), jnp.int32),
                   jax.ShapeDtypeStruct((EB, 128), jnp.int32)),
    )(t2, s2, d2)

    M = 2 * E
    K = 32
    CT = -(-M // (NW * K))
    CT = CT + (CT % 2)
    MP = CT * NW * K
    pad = MP - M
    g_all = jnp.concatenate(
        [g1.reshape(-1), g2.reshape(-1), jnp.zeros((pad,), jnp.int32)])
    s_all = jnp.concatenate(
        [edge_index[1], edge_index[0], jnp.full((pad,), N, jnp.int32)])

    ZC = 128
    OUTR = -(-(N + 1) // (NS * 8)) * 8
    NACC = NS * OUTR

    mesh = plsc.VectorSubcoreMesh(
        core_axis_name="c", subcore_axis_name="s",
        num_cores=NC, num_subcores=NS)

    @functools.partial(
        pl.kernel,
        out_type=jax.ShapeDtypeStruct((NC, NACC, H), jnp.float32),
        mesh=mesh,
        scratch_types=[
            pltpu.VMEM((K,), jnp.int32),
            pltpu.VMEM((K,), jnp.int32),
            pltpu.VMEM((K,), jnp.int32),
            pltpu.VMEM((K,), jnp.int32),
            pltpu.VMEM((K, H), jnp.float32),
            pltpu.VMEM((K, H), jnp.float32),
            pltpu.VMEM_SHARED((NACC, H), jnp.float32),
            pltpu.SemaphoreType.DMA,
            pltpu.SemaphoreType.DMA,
        ],
    )
    def sc_scatter(y_hbm, g_hbm, s_hbm, z_hbm, out_hbm,
                   gbuf0, sbuf0, gbuf1, sbuf1, rows0, rows1, acc,
                   gsem0, gsem1):
        cid = lax.axis_index("c")
        sid = lax.axis_index("s")
        wid = cid * NS + sid

        for z in range(OUTR // ZC):
            pltpu.sync_copy(z_hbm, acc.at[pl.ds(sid * OUTR + z * ZC, ZC), :])
        zr = OUTR % ZC
        if zr:
            pltpu.sync_copy(
                z_hbm.at[pl.ds(0, zr)],
                acc.at[pl.ds(sid * OUTR + (OUTR // ZC) * ZC, zr), :])
        plsc.subcore_barrier()

        def idx_copy(c, gb, sb):
            base = (c * NW + wid) * K
            pltpu.sync_copy(g_hbm.at[pl.ds(base, K)], gb)
            pltpu.sync_copy(s_hbm.at[pl.ds(base, K)], sb)

        def g_start(gb, rb, sem):
            pltpu.async_copy(y_hbm.at[gb], rb, sem)

        def g_wait(rb, sem):
            pltpu.make_async_copy(y_hbm.at[pl.ds(0, K)], rb, sem).wait()

        def scat(rb, sb):
            pltpu.sync_copy(rb, acc.at[sb], add=True)

        idx_copy(0, gbuf0, sbuf0)
        g_start(gbuf0, rows0, gsem0)

        def pair(j, carry):
            c = 2 * j
            idx_copy(c + 1, gbuf1, sbuf1)
            g_start(gbuf1, rows1, gsem1)
            g_wait(rows0, gsem0)
            scat(rows0, sbuf0)
            idx_copy(c + 2, gbuf0, sbuf0)
            g_start(gbuf0, rows0, gsem0)
            g_wait(rows1, gsem1)
            scat(rows1, sbuf1)
            return carry

        lax.fori_loop(0, (CT - 2) // 2, pair, 0)

        idx_copy(CT - 1, gbuf1, sbuf1)
        g_start(gbuf1, rows1, gsem1)
        g_wait(rows0, gsem0)
        scat(rows0, sbuf0)
        g_wait(rows1, gsem1)
        scat(rows1, sbuf1)

        plsc.subcore_barrier()

        pltpu.sync_copy(
            acc.at[pl.ds(sid * OUTR, OUTR), :],
            out_hbm.at[cid, pl.ds(sid * OUTR, OUTR), :])

    partials = sc_scatter(
        y_flat, g_all, s_all, jnp.zeros((ZC, H), jnp.float32))

    out = pl.pallas_call(
        _combine_body,
        grid=(NB,),
        in_specs=[
            pl.BlockSpec((1, BN, H), lambda i: (0, i, 0)),
            pl.BlockSpec((1, BN, H), lambda i: (1, i, 0)),
        ],
        out_specs=pl.BlockSpec((BN, H), lambda i: (i, 0)),
        out_shape=jax.ShapeDtypeStruct((N, H), jnp.float32),
    )(partials, partials)
    return out


# speedup vs baseline: 1.2782x; 1.2782x over previous
import functools

import jax
import jax.numpy as jnp
from jax import lax
from jax.experimental import pallas as pl
from jax.experimental.pallas import tpu as pltpu
from jax.experimental.pallas import tpu_sc as plsc

NC = 2
NS = 16
LANES = 16
NW = NC * NS


def _table_body(x_ref, w_ref, y_ref):
    y_ref[0] = jnp.dot(x_ref[...], w_ref[0], preferred_element_type=jnp.float32)


def _index_body(t_ref, s_ref, d_ref, g1_ref, g2_ref, *, n, r):
    t = t_ref[...]
    g1_ref[...] = t * n + s_ref[...]
    g2_ref[...] = (t + r) * n + d_ref[...]


def _combine_body(a_ref, b_ref, o_ref):
    o_ref[...] = a_ref[0] + b_ref[0]


def kernel(edge_index, edge_type, embeddings, weights, biases):
    N, H = embeddings.shape
    L = weights.shape[0]
    RD = weights.shape[1]
    R = RD // 2
    E = edge_type.shape[0]

    w_last = weights[L - 1]
    b_last = biases[L - 1]
    KA = H + 8
    x_aug = jnp.concatenate(
        [embeddings,
         jnp.ones((N, 1), jnp.float32),
         jnp.zeros((N, KA - H - 1), jnp.float32)], axis=1)
    w_aug = jnp.concatenate(
        [w_last,
         b_last[:, None, :],
         jnp.zeros((RD, KA - H - 1, H), jnp.float32)], axis=1)

    NB = 10
    BN = N // NB
    y = pl.pallas_call(
        _table_body,
        grid=(RD, NB),
        in_specs=[
            pl.BlockSpec((BN, KA), lambda rd, nb: (nb, 0)),
            pl.BlockSpec((1, KA, H), lambda rd, nb: (rd, 0, 0)),
        ],
        out_specs=pl.BlockSpec((1, BN, H), lambda rd, nb: (rd, nb, 0)),
        out_shape=jax.ShapeDtypeStruct((RD, N, H), jnp.float32),
    )(x_aug, w_aug)
    y_flat = y.reshape(RD * N, H)

    EB = E // 128
    t2 = edge_type.reshape(EB, 128)
    s2 = edge_index[0].reshape(EB, 128)
    d2 = edge_index[1].reshape(EB, 128)
    g1, g2 = pl.pallas_call(
        functools.partial(_index_body, n=N, r=R),
        out_shape=(jax.ShapeDtypeStruct((EB, 128), jnp.int32),
                   jax.ShapeDtypeStruct((EB, 128), jnp.int32)),
    )(t2, s2, d2)

    M = 2 * E
    K = 96
    CT = -(-M // (NW * K))
    CT = CT + (CT % 2)
    MP = CT * NW * K
    pad = MP - M
    g_all = jnp.concatenate(
        [g1.reshape(-1), g2.reshape(-1), jnp.zeros((pad,), jnp.int32)])
    s_all = jnp.concatenate(
        [edge_index[1], edge_index[0], jnp.full((pad,), N, jnp.int32)])

    ZC = 128
    OUTR = -(-(N + 1) // (NS * 8)) * 8
    NACC = NS * OUTR

    mesh = plsc.VectorSubcoreMesh(
        core_axis_name="c", subcore_axis_name="s",
        num_cores=NC, num_subcores=NS)

    @functools.partial(
        pl.kernel,
        out_type=jax.ShapeDtypeStruct((NC, NACC, H), jnp.float32),
        mesh=mesh,
        scratch_types=[
            pltpu.VMEM((K,), jnp.int32),
            pltpu.VMEM((K,), jnp.int32),
            pltpu.VMEM((K,), jnp.int32),
            pltpu.VMEM((K,), jnp.int32),
            pltpu.VMEM((K, H), jnp.float32),
            pltpu.VMEM((K, H), jnp.float32),
            pltpu.VMEM_SHARED((NACC, H), jnp.float32),
            pltpu.SemaphoreType.DMA,
            pltpu.SemaphoreType.DMA,
        ],
    )
    def sc_scatter(y_hbm, g_hbm, s_hbm, z_hbm, out_hbm,
                   gbuf0, sbuf0, gbuf1, sbuf1, rows0, rows1, acc,
                   gsem0, gsem1):
        cid = lax.axis_index("c")
        sid = lax.axis_index("s")
        wid = cid * NS + sid

        for z in range(OUTR // ZC):
            pltpu.sync_copy(z_hbm, acc.at[pl.ds(sid * OUTR + z * ZC, ZC), :])
        zr = OUTR % ZC
        if zr:
            pltpu.sync_copy(
                z_hbm.at[pl.ds(0, zr)],
                acc.at[pl.ds(sid * OUTR + (OUTR // ZC) * ZC, zr), :])
        plsc.subcore_barrier()

        def idx_copy(c, gb, sb):
            base = (c * NW + wid) * K
            pltpu.sync_copy(g_hbm.at[pl.ds(base, K)], gb)
            pltpu.sync_copy(s_hbm.at[pl.ds(base, K)], sb)

        def g_start(gb, rb, sem):
            pltpu.async_copy(y_hbm.at[gb], rb, sem)

        def g_wait(rb, sem):
            pltpu.make_async_copy(y_hbm.at[pl.ds(0, K)], rb, sem).wait()

        def scat(rb, sb):
            pltpu.sync_copy(rb, acc.at[sb], add=True)

        idx_copy(0, gbuf0, sbuf0)
        g_start(gbuf0, rows0, gsem0)

        def pair(j, carry):
            c = 2 * j
            idx_copy(c + 1, gbuf1, sbuf1)
            g_start(gbuf1, rows1, gsem1)
            g_wait(rows0, gsem0)
            scat(rows0, sbuf0)
            idx_copy(c + 2, gbuf0, sbuf0)
            g_start(gbuf0, rows0, gsem0)
            g_wait(rows1, gsem1)
            scat(rows1, sbuf1)
            return carry

        lax.fori_loop(0, (CT - 2) // 2, pair, 0)

        idx_copy(CT - 1, gbuf1, sbuf1)
        g_start(gbuf1, rows1, gsem1)
        g_wait(rows0, gsem0)
        scat(rows0, sbuf0)
        g_wait(rows1, gsem1)
        scat(rows1, sbuf1)

        plsc.subcore_barrier()

        pltpu.sync_copy(
            acc.at[pl.ds(sid * OUTR, OUTR), :],
            out_hbm.at[cid, pl.ds(sid * OUTR, OUTR), :])

    partials = sc_scatter(
        y_flat, g_all, s_all, jnp.zeros((ZC, H), jnp.float32))

    out = pl.pallas_call(
        _combine_body,
        grid=(NB,),
        in_specs=[
            pl.BlockSpec((1, BN, H), lambda i: (0, i, 0)),
            pl.BlockSpec((1, BN, H), lambda i: (1, i, 0)),
        ],
        out_specs=pl.BlockSpec((BN, H), lambda i: (i, 0)),
        out_shape=jax.ShapeDtypeStruct((N, H), jnp.float32),
    )(partials, partials)
    return out
